# baseline (speedup 1.0000x reference)
import jax
import jax.numpy as jnp
from jax.experimental import pallas as pl
from jax.experimental.pallas import tpu as pltpu

D_MODEL = 32
CLIP = 128
NUM_EMB = 2 * CLIP + 1
LQ = 2048
LKV = 2048
E2_ROWS = 4096
NSEM = 8


def _row_copy(g, e2_ref, out_ref, sem_ref):
    return pltpu.make_async_copy(
        e2_ref.at[pl.ds(LQ - g, LKV), :],
        out_ref.at[g],
        sem_ref.at[jax.lax.rem(g, NSEM)],
    )


def _band_kernel(e2_ref, out_ref, sem_ref):
    g = pl.program_id(0)
    _row_copy(g, e2_ref, out_ref, sem_ref).start()

    @pl.when(g >= NSEM)
    def _wait_prev():
        _row_copy(g - NSEM, e2_ref, out_ref, sem_ref).wait()

    @pl.when(g == LQ - 1)
    def _drain():
        for d in range(NSEM):
            _row_copy(LQ - NSEM + d, e2_ref, out_ref, sem_ref).wait()


def kernel(length_q, length_kv, embedding):
    del length_q, length_kv
    top = jnp.broadcast_to(embedding[0:1, :], (1920, D_MODEL))
    bot = jnp.broadcast_to(embedding[NUM_EMB - 1:NUM_EMB, :], (1920, D_MODEL))
    e2 = jnp.concatenate([top, embedding[0:256, :], bot], axis=0)
    return pl.pallas_call(
        _band_kernel,
        grid=(LQ,),
        in_specs=[pl.BlockSpec(memory_space=pltpu.MemorySpace.HBM)],
        out_specs=pl.BlockSpec(memory_space=pltpu.MemorySpace.HBM),
        out_shape=jax.ShapeDtypeStruct((LQ, LKV, D_MODEL), jnp.float32),
        scratch_shapes=[pltpu.SemaphoreType.DMA((NSEM,))],
    )(e2)


# speedup vs baseline: 47.3393x; 47.3393x over previous
import jax
import jax.numpy as jnp
from jax.experimental import pallas as pl
from jax.experimental.pallas import tpu as pltpu

D_MODEL = 32
CLIP = 128
NUM_EMB = 2 * CLIP + 1
LQ = 2048
LKV = 2048
ROW128 = LKV * D_MODEL // 128
E2S_ROWS = 4096 * D_MODEL // 128
BQ = 8
NCHUNK = 4
CROWS = LQ // NCHUNK


def _make_body(row0):
    def _body(e2s_ref, out_ref):
        base = row0 + pl.program_id(0) * BQ
        for r in range(BQ):
            g = base + r
            c = (-r) % 4
            q = (LQ - g) // 4
            out_ref[r] = e2s_ref[c, pl.ds(q, ROW128), :]

    return _body


def kernel(length_q, length_kv, embedding):
    del length_q, length_kv
    top = jnp.broadcast_to(embedding[0:1, :], (1920, D_MODEL))
    bot = jnp.broadcast_to(embedding[NUM_EMB - 1:NUM_EMB, :], (1920, D_MODEL))
    flat = jnp.concatenate([top, embedding[0:256, :], bot], axis=0).reshape(-1)
    flat = jnp.concatenate([flat, jnp.zeros((96,), jnp.float32)])
    e2s = jnp.stack(
        [flat[32 * c:32 * c + E2S_ROWS * 128].reshape(E2S_ROWS, 128) for c in range(4)]
    )
    chunks = []
    for ci in range(NCHUNK):
        out = pl.pallas_call(
            _make_body(ci * CROWS),
            grid=(CROWS // BQ,),
            in_specs=[pl.BlockSpec((4, E2S_ROWS, 128), lambda i: (0, 0, 0))],
            out_specs=pl.BlockSpec((BQ, ROW128, 128), lambda i: (i, 0, 0)),
            out_shape=jax.ShapeDtypeStruct((CROWS, ROW128, 128), jnp.float32),
        )(e2s)
        chunks.append(out.reshape(CROWS, LKV, D_MODEL))
    return jnp.concatenate(chunks, axis=0)
